# baseline (speedup 1.0000x reference)
import functools

import jax
import jax.numpy as jnp
from jax import lax
from jax.experimental import pallas as pl
from jax.experimental.pallas import tpu as pltpu
from jax.experimental.pallas import tpu_sc as plsc

BATCH = 16384
DIM = 64
LANES = 16
NUM_CORES = 2
NUM_SUBCORES = 16
NUM_WORKERS = NUM_CORES * NUM_SUBCORES
B_PER_W = BATCH // NUM_WORKERS
HALF = B_PER_W // 2
IDX_W = 128
GROUPS = HALF // LANES


def _body(uids_hbm, iids_hbm, up_hbm, ip_hbm, out_hbm,
          ids_u, ids_v, blk_u, blk_v, rows_u, rows_v, out_v, sem):
    w = lax.axis_index("s") * NUM_CORES + lax.axis_index("c")
    base = w * B_PER_W

    pltpu.sync_copy(uids_hbm.at[pl.ds(base, B_PER_W)], ids_u)
    pltpu.sync_copy(iids_hbm.at[pl.ds(base, B_PER_W)], ids_v)

    for k in range(B_PER_W // LANES):
        iu = ids_u[pl.ds(k * LANES, LANES)]
        iv = ids_v[pl.ds(k * LANES, LANES)]
        blk_u[k // 8, pl.ds((k % 8) * LANES, LANES)] = jnp.right_shift(iu, 1)
        blk_v[k // 8, pl.ds((k % 8) * LANES, LANES)] = jnp.right_shift(iv, 1)

    lane = lax.iota(jnp.int32, LANES)

    for h in range(2):
        for k in range(2):
            pltpu.async_copy(up_hbm.at[blk_u.at[h * 2 + k]],
                             rows_u.at[pl.ds(k * IDX_W, IDX_W)], sem)
            pltpu.async_copy(ip_hbm.at[blk_v.at[h * 2 + k]],
                             rows_v.at[pl.ds(k * IDX_W, IDX_W)], sem)
        pltpu.make_async_copy(up_hbm.at[blk_u.at[0]], rows_u, sem).wait()
        pltpu.make_async_copy(ip_hbm.at[blk_v.at[0]], rows_v, sem).wait()

        def group(g, _):
            row_idx = g * LANES + lane
            idu = ids_u[pl.ds(h * HALF + g * LANES, LANES)]
            idv = ids_v[pl.ds(h * HALF + g * LANES, LANES)]
            cu0 = jnp.left_shift(jnp.bitwise_and(idu, 1), 6)
            cv0 = jnp.left_shift(jnp.bitwise_and(idv, 1), 6)
            acc = jnp.zeros((LANES,), jnp.float32)
            for d in range(DIM):
                u = plsc.load_gather(rows_u, [row_idx, cu0 + d])
                v = plsc.load_gather(rows_v, [row_idx, cv0 + d])
                acc = acc + u * v
            out_v[pl.ds(h * HALF + g * LANES, LANES)] = acc
            return 0

        lax.fori_loop(0, GROUPS, group, 0)

    pltpu.sync_copy(out_v, out_hbm.at[pl.ds(base, B_PER_W)])


def kernel(user_ids, item_ids, user_table, item_table):
    nu, dim = user_table.shape
    ni = item_table.shape[0]
    up = user_table.reshape(nu // 2, 2 * dim)
    ip = item_table.reshape(ni // 2, 2 * dim)
    uids = user_ids.astype(jnp.int32)
    iids = item_ids.astype(jnp.int32)

    mesh = plsc.VectorSubcoreMesh(
        core_axis_name="c", subcore_axis_name="s",
        num_cores=NUM_CORES, num_subcores=NUM_SUBCORES)

    run = pl.kernel(
        _body,
        out_type=jax.ShapeDtypeStruct((BATCH,), jnp.float32),
        mesh=mesh,
        scratch_types=[
            pltpu.VMEM((B_PER_W,), jnp.int32),
            pltpu.VMEM((B_PER_W,), jnp.int32),
            pltpu.VMEM((4, IDX_W), jnp.int32),
            pltpu.VMEM((4, IDX_W), jnp.int32),
            pltpu.VMEM((HALF, 2 * DIM), jnp.float32),
            pltpu.VMEM((HALF, 2 * DIM), jnp.float32),
            pltpu.VMEM((B_PER_W,), jnp.float32),
            pltpu.SemaphoreType.DMA,
        ],
        compiler_params=pltpu.CompilerParams(
            needs_layout_passes=False, use_tc_tiling_on_sc=True),
    )
    return run(uids, iids, up, ip)
